# baseline (speedup 1.0000x reference)
import functools

import jax
import jax.numpy as jnp
from jax import lax
from jax.experimental import pallas as pl
from jax.experimental.pallas import tpu as pltpu
from jax.experimental.pallas import tpu_sc as plsc

_B, _S, _N = 8, 1024, 32
_HID = 128
_L = 16
_NC, _NS = 2, 16
_NW = _NC * _NS
_NRG = 16
_HCOL = _HID // 2
_GROWS = _B * _S // _NRG
_GSLOTS = _GROWS * _N
_TROWS = 9 * 1024
_LT_ROWS = _S + 2

_GDN = lax.GatherDimensionNumbers(
    offset_dims=(), collapsed_slice_dims=(0,), start_index_map=(0,))


def _take16(vec, idx):
    return lax.gather(vec, idx[:, None], dimension_numbers=_GDN,
                      slice_sizes=(1,),
                      mode=lax.GatherScatterMode.PROMISE_IN_BOUNDS)


def _build_table(h, ptail, pt0, Wn, idx2d, msk2d):
    def body(h_ref, ptail_ref, pt0_ref, wn_ref, idx_ref, msk_ref,
             tbl_ref, gidx_ref):
        ws = wn_ref[...] * (1.0 / _N)
        t = (h_ref[...] + ptail_ref[...][None]).reshape(_B * _S, _HID)
        rows = lax.broadcasted_iota(jnp.int32, (_S, _HID), 0)
        special = jnp.where(rows == 0, pt0_ref[...], 0.0)
        full = jnp.concatenate([t, special], axis=0)
        res = lax.dot_general(full, ws, (((1,), (1,)), ((), ())),
                              preferred_element_type=jnp.float32)
        tbl_ref[0] = res[:, :_HCOL]
        tbl_ref[1] = res[:, _HCOL:]
        v = idx_ref[...]
        gidx_ref[...] = jnp.where(msk_ref[...],
                                  jnp.where(v == 0, _S, v - 1),
                                  _S + 1)

    return pl.pallas_call(
        body,
        out_shape=(
            jax.ShapeDtypeStruct((2, _TROWS, _HCOL), jnp.float32),
            jax.ShapeDtypeStruct(idx2d.shape, jnp.int32),
        ),
    )(h, ptail, pt0, Wn, idx2d, msk2d)


@functools.lru_cache(maxsize=1)
def _make_sc_gather():
    mesh = plsc.VectorSubcoreMesh(core_axis_name="c", subcore_axis_name="s")

    @functools.partial(
        pl.kernel,
        mesh=mesh,
        compiler_params=pltpu.CompilerParams(
            needs_layout_passes=False, use_tc_tiling_on_sc=False),
        out_type=jax.ShapeDtypeStruct((2, _B * _S, _HCOL), jnp.float32),
        scratch_types=[
            pltpu.VMEM((_LT_ROWS, _HCOL), jnp.float32),
            pltpu.VMEM((_GSLOTS,), jnp.int32),
            pltpu.VMEM((_GROWS, _HCOL), jnp.float32),
            pltpu.SemaphoreType.DMA,
            pltpu.SemaphoreType.DMA,
        ],
    )
    def sc_gather(t_hbm, gidx_hbm, out_hbm, ttile, idx_v, oacc, sem_t, sem_i):
        wid = lax.axis_index("s") * _NC + lax.axis_index("c")
        rg = wid % _NRG
        half = wid // _NRG
        b = rg // (_NRG // _B)
        sbase = rg * _GSLOTS

        pltpu.async_copy(t_hbm.at[half, pl.ds(b * _S, _S)],
                         ttile.at[pl.ds(0, _S)], sem_t)
        pltpu.async_copy(t_hbm.at[half, pl.ds(_B * _S, 2)],
                         ttile.at[pl.ds(_S, 2)], sem_t)
        pltpu.async_copy(gidx_hbm.at[pl.ds(sbase, _GSLOTS)], idx_v, sem_i)

        coffs = [jnp.arange(_L, dtype=jnp.int32) + c * _L
                 for c in range(_HCOL // _L)]
        lane_consts = [jnp.full((_L,), n, dtype=jnp.int32) for n in range(_L)]
        zero = jnp.zeros((_L,), jnp.float32)

        pltpu.make_async_copy(gidx_hbm.at[pl.ds(sbase, _GSLOTS)], idx_v,
                              sem_i).wait()
        pltpu.make_async_copy(
            t_hbm.at[half, pl.ds(0, _LT_ROWS)], ttile, sem_t).wait()

        def row_body(s, _):
            for hblk in range(2):
                iv = idx_v[pl.ds(s * _N + hblk * _L, _L)]
                for n in range(_L):
                    spl = _take16(iv, lane_consts[n])
                    for c in range(_HCOL // _L):
                        val = plsc.load_gather(ttile, [spl, coffs[c]])
                        if hblk == 0 and n == 0:
                            oacc[s, pl.ds(c * _L, _L)] = val
                        else:
                            plsc.addupdate(oacc.at[s, pl.ds(c * _L, _L)],
                                           val)
            return 0

        lax.fori_loop(0, _GROWS, row_body, 0)
        pltpu.sync_copy(oacc, out_hbm.at[half, pl.ds(rg * _GROWS, _GROWS)])

    return sc_gather


def kernel(x, h, g, neighbor_index, neighbor_mask, Wn, pos_table):
    del x, g
    idx2d = neighbor_index.astype(jnp.int32).reshape(_B * _S * _N // _HID,
                                                     _HID)
    msk2d = neighbor_mask.reshape(_B * _S * _N // _HID, _HID)
    table, gidx = _build_table(h, pos_table[1:], pos_table[0:1], Wn,
                               idx2d, msk2d)
    halves = _make_sc_gather()(table, gidx.reshape(-1))
    return jnp.concatenate([halves[0], halves[1]], axis=-1).reshape(
        _B, _S, _HID)


# speedup vs baseline: 2.7207x; 2.7207x over previous
import functools

import jax
import jax.numpy as jnp
from jax import lax
from jax.experimental import pallas as pl
from jax.experimental.pallas import tpu as pltpu
from jax.experimental.pallas import tpu_sc as plsc

_B, _S, _N = 8, 1024, 32
_HID = 128
_L = 16
_NC, _NS = 2, 16
_NW = _NC * _NS
_NRG = 16
_HCOL = _HID // 2
_GROWS = _B * _S // _NRG
_GSLOTS = _GROWS * _N
_TROWS = 9 * 1024
_LT_ROWS = _S + 2

_GDN = lax.GatherDimensionNumbers(
    offset_dims=(), collapsed_slice_dims=(0,), start_index_map=(0,))


def _take16(vec, idx):
    return lax.gather(vec, idx[:, None], dimension_numbers=_GDN,
                      slice_sizes=(1,),
                      mode=lax.GatherScatterMode.PROMISE_IN_BOUNDS)


def _build_table(h, ptail, pt0, Wn, idx2d, msk2d):
    def body(h_ref, ptail_ref, pt0_ref, wn_ref, idx_ref, msk_ref,
             tbl_ref, gidx_ref):
        ws = wn_ref[...] * (1.0 / _N)
        t = (h_ref[...] + ptail_ref[...][None]).reshape(_B * _S, _HID)
        rows = lax.broadcasted_iota(jnp.int32, (_S, _HID), 0)
        special = jnp.where(rows == 0, pt0_ref[...], 0.0)
        full = jnp.concatenate([t, special], axis=0)
        res = lax.dot_general(full, ws, (((1,), (1,)), ((), ())),
                              preferred_element_type=jnp.float32)
        tbl_ref[0] = res[:, :_HCOL]
        tbl_ref[1] = res[:, _HCOL:]
        v = idx_ref[...]
        gidx_ref[...] = jnp.where(msk_ref[...],
                                  jnp.where(v == 0, _S, v - 1),
                                  _S + 1)

    return pl.pallas_call(
        body,
        out_shape=(
            jax.ShapeDtypeStruct((2, _TROWS, _HCOL), jnp.float32),
            jax.ShapeDtypeStruct((_B * _S * _N // _HID, _HID), jnp.int32),
        ),
    )(h, ptail, pt0, Wn, idx2d, msk2d)


@functools.lru_cache(maxsize=1)
def _make_sc_gather():
    mesh = plsc.VectorSubcoreMesh(core_axis_name="c", subcore_axis_name="s")

    @functools.partial(
        pl.kernel,
        mesh=mesh,
        compiler_params=pltpu.CompilerParams(
            needs_layout_passes=False, use_tc_tiling_on_sc=False),
        out_type=jax.ShapeDtypeStruct((2, _B * _S, _HCOL), jnp.float32),
        scratch_types=[
            pltpu.VMEM((_LT_ROWS, _HCOL), jnp.float32),
            pltpu.VMEM((_GSLOTS,), jnp.int32),
            pltpu.VMEM((_GROWS, _HCOL), jnp.float32),
            pltpu.SemaphoreType.DMA,
            pltpu.SemaphoreType.DMA,
        ],
    )
    def sc_gather(t_hbm, gidx_hbm, out_hbm, ttile, idx_v, oacc, sem_t, sem_i):
        wid = lax.axis_index("s") * _NC + lax.axis_index("c")
        rg = wid % _NRG
        half = wid // _NRG
        b = rg // (_NRG // _B)
        sbase = rg * _GSLOTS

        pltpu.async_copy(t_hbm.at[half, pl.ds(b * _S, _S)],
                         ttile.at[pl.ds(0, _S)], sem_t)
        pltpu.async_copy(t_hbm.at[half, pl.ds(_B * _S, 2)],
                         ttile.at[pl.ds(_S, 2)], sem_t)
        pltpu.async_copy(gidx_hbm.at[pl.ds(sbase, _GSLOTS)], idx_v, sem_i)

        coffs = [jnp.arange(_L, dtype=jnp.int32) + c * _L
                 for c in range(_HCOL // _L)]
        lane_consts = [jnp.full((_L,), n, dtype=jnp.int32) for n in range(_L)]
        zero = jnp.zeros((_L,), jnp.float32)

        pltpu.make_async_copy(gidx_hbm.at[pl.ds(sbase, _GSLOTS)], idx_v,
                              sem_i).wait()
        pltpu.make_async_copy(
            t_hbm.at[half, pl.ds(0, _LT_ROWS)], ttile, sem_t).wait()

        def row_body(s, _):
            accs = [zero for _ in range(_HCOL // _L)]
            for hblk in range(2):
                iv = idx_v[pl.ds(s * _N + hblk * _L, _L)]
                for n in range(_L):
                    spl = _take16(iv, lane_consts[n])
                    for c in range(_HCOL // _L):
                        val = plsc.load_gather(ttile, [spl, coffs[c]])
                        accs[c] = accs[c] + val
            for c in range(_HCOL // _L):
                oacc[s, pl.ds(c * _L, _L)] = accs[c]
            return 0

        lax.fori_loop(0, _GROWS, row_body, 0)
        pltpu.sync_copy(oacc, out_hbm.at[half, pl.ds(rg * _GROWS, _GROWS)])

    return sc_gather


def kernel(x, h, g, neighbor_index, neighbor_mask, Wn, pos_table):
    del x, g
    idx2d = neighbor_index.astype(jnp.int32).reshape(_B * _S * _N // _HID,
                                                     _HID)
    msk2d = neighbor_mask.reshape(_B * _S * _N // _HID, _HID)
    table, gidx = _build_table(h, pos_table[1:], pos_table[0:1], Wn,
                               idx2d, msk2d)
    halves = _make_sc_gather()(table, gidx.reshape(-1))
    return jnp.concatenate([halves[0], halves[1]], axis=-1).reshape(
        _B, _S, _HID)
